# Initial kernel scaffold; baseline (speedup 1.0000x reference)
import jax
import jax.numpy as jnp
from jax.experimental import pallas as pl


def kernel(edge_index, edge_type, node_emb, W1, root1, b1, W2, root2, b2):
    raise NotImplementedError("write your pallas kernel here")


# speedup vs baseline: 2.6869x; 2.6869x over previous
import functools

import jax
import jax.numpy as jnp
from jax import lax
from jax.experimental import pallas as pl
from jax.experimental.pallas import tpu as pltpu
from jax.experimental.pallas import tpu_sc as plsc

N = 10000
E = 320000
D = 128
R = 32
B = 4
BS = D // B

HD = D // 2
NS = 16
CH = 250
K = 80
RP = R + 1
NT = 2000
NPAD = 10240


def _blockdiag(W, root):
    Wd = jnp.zeros((R, D, D), W.dtype)
    for b in range(B):
        Wd = Wd.at[:, b * BS:(b + 1) * BS, b * BS:(b + 1) * BS].set(W[:, b])
    return jnp.concatenate([Wd, root[None]], axis=0)


def _h_body(x_ref, w_ref, h_ref):
    h_ref[0] = jnp.dot(x_ref[...], w_ref[0], preferred_element_type=jnp.float32)


def _transform(x, Wall):
    return pl.pallas_call(
        _h_body,
        grid=(RP, N // NT),
        in_specs=[
            pl.BlockSpec((NT, D), lambda r, n: (n, 0)),
            pl.BlockSpec((1, D, D), lambda r, n: (r, 0, 0)),
        ],
        out_specs=pl.BlockSpec((1, NT, D), lambda r, n: (r, n, 0)),
        out_shape=jax.ShapeDtypeStruct((RP, N, D), jnp.float32),
    )(x, Wall)


def _sc_body(with_deg, table, srcs, rts, dsts, zrow, zdeg, agg_out, deg_out,
             src_v, idx_v, dst_v, rows_v, ones_v, agg_sh, deg_sh, sem):
    c = lax.axis_index("c")
    s = lax.axis_index("s")

    rows_per = NPAD // NS
    zsl = pl.ds(s * rows_per, rows_per)
    pltpu.sync_copy(zrow.at[zsl], agg_sh.at[zsl])
    if with_deg:
        pltpu.sync_copy(zdeg.at[zsl], deg_sh.at[zsl])

    pltpu.sync_copy(srcs.at[s], src_v)
    pltpu.sync_copy(rts.at[s], idx_v)
    pltpu.sync_copy(dsts.at[s], dst_v)

    if with_deg:
        def ones_body(i, _):
            ones_v[i] = jnp.full((16,), 1.0, jnp.float32)
            return 0
        lax.fori_loop(0, K, ones_body, 0)

    def idx_body(j, _):
        for i in range(K // 16):
            sl = pl.ds(i * 16, 16)
            idx_v[j, sl] = (idx_v[j, sl] * N + src_v[j, sl]) * 2 + c
        return 0
    lax.fori_loop(0, CH, idx_body, 0)

    plsc.subcore_barrier()

    def chunk_body(j, _):
        pltpu.async_copy(table.at[idx_v.at[j]], rows_v, sem).wait()
        pltpu.sync_copy(rows_v, agg_sh.at[dst_v.at[j]], add=True)
        if with_deg:
            @pl.when(c == 0)
            def _():
                pltpu.sync_copy(ones_v, deg_sh.at[dst_v.at[j]], add=True)
        return 0
    lax.fori_loop(0, CH, chunk_body, 0)

    plsc.subcore_barrier()

    pltpu.sync_copy(agg_sh.at[zsl], agg_out.at[c, zsl])
    if with_deg:
        @pl.when(c == 0)
        def _():
            pltpu.sync_copy(deg_sh.at[zsl], deg_out.at[zsl])


def _sc_gather_scatter(table, srcs, rts, dsts, with_deg):
    mesh = plsc.VectorSubcoreMesh(core_axis_name="c", subcore_axis_name="s")
    out_type = (jax.ShapeDtypeStruct((2, NPAD, HD), jnp.float32),
                jax.ShapeDtypeStruct((NPAD, 16), jnp.float32))
    scratch = [
        pltpu.VMEM((CH, K), jnp.int32),
        pltpu.VMEM((CH, K), jnp.int32),
        pltpu.VMEM((CH, K), jnp.int32),
        pltpu.VMEM((K, HD), jnp.float32),
        pltpu.VMEM((K, 16), jnp.float32),
        pltpu.VMEM_SHARED((NPAD, HD), jnp.float32),
        pltpu.VMEM_SHARED((NPAD, 16), jnp.float32) if with_deg else None,
        pltpu.SemaphoreType.DMA,
    ]
    if not with_deg:
        scratch.pop(6)
    zrow = jnp.zeros((NPAD, HD), jnp.float32)
    zdeg = jnp.zeros((NPAD, 16), jnp.float32)

    if with_deg:
        body = functools.partial(_sc_body, True)
    else:
        def body(table, srcs, rts, dsts, zrow, zdeg, agg_out, deg_out,
                 src_v, idx_v, dst_v, rows_v, ones_v, agg_sh, sem):
            _sc_body(False, table, srcs, rts, dsts, zrow, zdeg, agg_out,
                     deg_out, src_v, idx_v, dst_v, rows_v, ones_v, agg_sh,
                     None, sem)
    k = pl.kernel(body, out_type=out_type, mesh=mesh, scratch_types=scratch,
                  compiler_params=pltpu.CompilerParams(
                      use_tc_tiling_on_sc=False))
    return k(table, srcs, rts, dsts, zrow, zdeg)


def _combine_body(relu, p_ref, dp_ref, rt_ref, b_ref, o_ref):
    agg = jnp.concatenate([p_ref[0], p_ref[1]], axis=-1)
    deg = dp_ref[:, 0:1]
    y = agg / jnp.maximum(deg, 1.0) + rt_ref[0] + b_ref[...]
    o_ref[...] = jnp.maximum(y, 0.0) if relu else y


def _combine(partials, degp, H, bias, relu):
    return pl.pallas_call(
        functools.partial(_combine_body, relu),
        grid=(N // NT,),
        in_specs=[
            pl.BlockSpec((2, NT, HD), lambda n: (0, n, 0)),
            pl.BlockSpec((NT, 16), lambda n: (n, 0)),
            pl.BlockSpec((1, NT, D), lambda n: (R, n, 0)),
            pl.BlockSpec((1, D), lambda n: (0, 0)),
        ],
        out_specs=pl.BlockSpec((NT, D), lambda n: (n, 0)),
        out_shape=jax.ShapeDtypeStruct((N, D), jnp.float32),
    )(partials, degp, H, bias.reshape(1, D))


def kernel(edge_index, edge_type, node_emb, W1, root1, b1, W2, root2, b2):
    srcs = edge_index[:, 0].reshape(NS, CH, K)
    dsts = edge_index[:, 1].reshape(NS, CH, K)
    rts = edge_type.reshape(NS, CH, K)

    Wall1 = _blockdiag(W1, root1)
    Wall2 = _blockdiag(W2, root2)

    H1 = _transform(node_emb, Wall1)
    agg1, degp = _sc_gather_scatter(H1.reshape(RP * N * 2, HD), srcs, rts,
                                    dsts, with_deg=True)
    x1 = _combine(agg1, degp, H1, b1, relu=True)

    H2 = _transform(x1, Wall2)
    agg2, _ = _sc_gather_scatter(H2.reshape(RP * N * 2, HD), srcs, rts, dsts,
                                 with_deg=False)
    return _combine(agg2, degp, H2, b2, relu=False)


# speedup vs baseline: 4.0053x; 1.4907x over previous
import functools

import jax
import jax.numpy as jnp
from jax import lax
from jax.experimental import pallas as pl
from jax.experimental.pallas import tpu as pltpu
from jax.experimental.pallas import tpu_sc as plsc

N = 10000
E = 320000
D = 128
R = 32
B = 4
BS = D // B

HD = D // 2
NS = 16
CH = 250
K = 80
RP = R + 1
NT = 2000
NPAD = 10240

_SC_PARAMS = pltpu.CompilerParams(use_tc_tiling_on_sc=False)


def _blockdiag(W, root):
    Wd = jnp.zeros((R, D, D), W.dtype)
    for b in range(B):
        Wd = Wd.at[:, b * BS:(b + 1) * BS, b * BS:(b + 1) * BS].set(W[:, b])
    return jnp.concatenate([Wd, root[None]], axis=0)


def _h_body(x_ref, w_ref, h_ref):
    h_ref[0] = jnp.dot(x_ref[...], w_ref[0], preferred_element_type=jnp.float32)


def _transform(x, Wall):
    return pl.pallas_call(
        _h_body,
        grid=(N // NT, RP),
        in_specs=[
            pl.BlockSpec((NT, D), lambda n, r: (n, 0)),
            pl.BlockSpec((1, D, D), lambda n, r: (r, 0, 0)),
        ],
        out_specs=pl.BlockSpec((1, NT, D), lambda n, r: (r, n, 0)),
        out_shape=jax.ShapeDtypeStruct((RP, N, D), jnp.float32),
    )(x, Wall)


def _deg_body(dsts, zdeg, deg_out, dst_v, ones_v, deg_sh):
    c = lax.axis_index("c")
    s = lax.axis_index("s")

    rows_per = NPAD // NS
    zsl = pl.ds(s * rows_per, rows_per)
    pltpu.sync_copy(zdeg.at[zsl], deg_sh.at[zsl])

    pltpu.sync_copy(dsts.at[s], dst_v)

    def ones_body(i, _):
        ones_v[i] = jnp.full((16,), 1.0, jnp.float32)
        return 0
    lax.fori_loop(0, K, ones_body, 0)

    plsc.subcore_barrier()

    half = CH // 2

    def chunk_body(j, _):
        pltpu.sync_copy(ones_v, deg_sh.at[dst_v.at[c * half + j]], add=True)
        return 0
    lax.fori_loop(0, half, chunk_body, 0)

    plsc.subcore_barrier()
    pltpu.sync_copy(deg_sh.at[zsl], deg_out.at[c, zsl])


def _sc_degrees(dsts):
    mesh = plsc.VectorSubcoreMesh(core_axis_name="c", subcore_axis_name="s")
    k = pl.kernel(
        _deg_body,
        out_type=jax.ShapeDtypeStruct((2, NPAD, 16), jnp.float32),
        mesh=mesh,
        scratch_types=[
            pltpu.VMEM((CH, K), jnp.int32),
            pltpu.VMEM((K, 16), jnp.float32),
            pltpu.VMEM_SHARED((NPAD, 16), jnp.float32),
        ],
        compiler_params=_SC_PARAMS,
    )
    return k(dsts, jnp.zeros((NPAD, 16), jnp.float32))


def _sc_body(table, srcs, rts, dsts, zrow, agg_out,
             src_v, idx_v, dst_v, rows0, rows1, agg_sh, sem):
    c = lax.axis_index("c")
    s = lax.axis_index("s")

    rows_per = NPAD // NS
    zsl = pl.ds(s * rows_per, rows_per)
    pltpu.sync_copy(zrow.at[zsl], agg_sh.at[zsl])

    pltpu.sync_copy(srcs.at[s], src_v)
    pltpu.sync_copy(rts.at[s], idx_v)
    pltpu.sync_copy(dsts.at[s], dst_v)

    def idx_body(j, _):
        for i in range(K // 16):
            sl = pl.ds(i * 16, 16)
            idx_v[j, sl] = (idx_v[j, sl] * N + src_v[j, sl]) * 2 + c
        return 0
    lax.fori_loop(0, CH, idx_body, 0)

    plsc.subcore_barrier()

    pltpu.async_copy(table.at[idx_v.at[0]], rows0, sem)

    def pair_body(t, _):
        j0 = 2 * t
        j1 = j0 + 1
        pltpu.async_copy(table.at[idx_v.at[j1]], rows1, sem)
        pltpu.make_async_copy(table.at[idx_v.at[j0]], rows0, sem).wait()
        pltpu.sync_copy(rows0, agg_sh.at[dst_v.at[j0]], add=True)

        @pl.when(j1 + 1 < CH)
        def _():
            pltpu.async_copy(table.at[idx_v.at[j1 + 1]], rows0, sem)
        pltpu.make_async_copy(table.at[idx_v.at[j1]], rows1, sem).wait()
        pltpu.sync_copy(rows1, agg_sh.at[dst_v.at[j1]], add=True)
        return 0
    lax.fori_loop(0, CH // 2, pair_body, 0)

    plsc.subcore_barrier()
    pltpu.sync_copy(agg_sh.at[zsl], agg_out.at[c, zsl])


def _sc_gather_scatter(table, srcs, rts, dsts):
    mesh = plsc.VectorSubcoreMesh(core_axis_name="c", subcore_axis_name="s")
    k = pl.kernel(
        _sc_body,
        out_type=jax.ShapeDtypeStruct((2, NPAD, HD), jnp.float32),
        mesh=mesh,
        scratch_types=[
            pltpu.VMEM((CH, K), jnp.int32),
            pltpu.VMEM((CH, K), jnp.int32),
            pltpu.VMEM((CH, K), jnp.int32),
            pltpu.VMEM((K, HD), jnp.float32),
            pltpu.VMEM((K, HD), jnp.float32),
            pltpu.VMEM_SHARED((NPAD, HD), jnp.float32),
            pltpu.SemaphoreType.DMA,
        ],
        compiler_params=_SC_PARAMS,
    )
    return k(table, srcs, rts, dsts, jnp.zeros((NPAD, HD), jnp.float32))


def _combine_body(relu, p_ref, dp_ref, rt_ref, b_ref, o_ref):
    agg = jnp.concatenate([p_ref[0], p_ref[1]], axis=-1)
    deg = dp_ref[0, :, 0:1] + dp_ref[1, :, 0:1]
    y = agg / jnp.maximum(deg, 1.0) + rt_ref[0] + b_ref[...]
    o_ref[...] = jnp.maximum(y, 0.0) if relu else y


def _combine(partials, degp, H, bias, relu):
    return pl.pallas_call(
        functools.partial(_combine_body, relu),
        grid=(N // NT,),
        in_specs=[
            pl.BlockSpec((2, NT, HD), lambda n: (0, n, 0)),
            pl.BlockSpec((2, NT, 16), lambda n: (0, n, 0)),
            pl.BlockSpec((1, NT, D), lambda n: (R, n, 0)),
            pl.BlockSpec((1, D), lambda n: (0, 0)),
        ],
        out_specs=pl.BlockSpec((NT, D), lambda n: (n, 0)),
        out_shape=jax.ShapeDtypeStruct((N, D), jnp.float32),
    )(partials, degp, H, bias.reshape(1, D))


def kernel(edge_index, edge_type, node_emb, W1, root1, b1, W2, root2, b2):
    srcs = edge_index[:, 0].reshape(NS, CH, K)
    dsts = edge_index[:, 1].reshape(NS, CH, K)
    rts = edge_type.reshape(NS, CH, K)

    Wall1 = _blockdiag(W1, root1)
    Wall2 = _blockdiag(W2, root2)

    degp = _sc_degrees(dsts)
    H1 = _transform(node_emb, Wall1)
    agg1 = _sc_gather_scatter(H1.reshape(RP * N * 2, HD), srcs, rts, dsts)
    x1 = _combine(agg1, degp, H1, b1, relu=True)

    H2 = _transform(x1, Wall2)
    agg2 = _sc_gather_scatter(H2.reshape(RP * N * 2, HD), srcs, rts, dsts)
    return _combine(agg2, degp, H2, b2, relu=False)
